# Initial kernel scaffold; baseline (speedup 1.0000x reference)
import jax
import jax.numpy as jnp
from jax.experimental import pallas as pl


def kernel(features, adj, pairs, sub_paths, sub_path_length, W_pw, W_ih, W_hh, b_ih, b_hh, w_out, b_out, W1, b1, W2, b2):
    raise NotImplementedError("write your pallas kernel here")


# speedup vs baseline: 1.6238x; 1.6238x over previous
import functools

import jax
import jax.numpy as jnp
from jax.experimental import pallas as pl
from jax.experimental.pallas import tpu as pltpu


def _pick_block(n, target, mult=8):
    best = None
    for b in range(mult, min(n, target) + 1, mult):
        if n % b == 0:
            best = b
    return best if best is not None else n


def _emb0_kernel(f_ref, w_ref, o_ref):
    o_ref[...] = jax.nn.relu(
        jnp.dot(f_ref[...], w_ref[...], preferred_element_type=jnp.float32))


def _spmv1_kernel(adj_ref, e_ref, x1_ref):
    x1_ref[...] = jnp.dot(adj_ref[...], e_ref[...],
                          preferred_element_type=jnp.float32)


def _spmv2_kernel(adj_ref, x1f_ref, e_blk_ref, x1_blk_ref, gnn_ref):
    x2 = jnp.dot(adj_ref[...], x1f_ref[...], preferred_element_type=jnp.float32)
    gnn_ref[...] = (e_blk_ref[...] + x1_blk_ref[...] + x2) * (1.0 / 3.0)


def _lstm_kernel(pe_ref, len_ref, wcat_ref, bias_ref, wout_ref, bout_ref,
                 pw_ref):
    Pb, Lb, _ = pe_ref.shape
    Hb = wcat_ref.shape[1] // 4
    h = jnp.zeros((Pb, Hb), jnp.float32)
    c = jnp.zeros((Pb, Hb), jnp.float32)
    hsel = jnp.zeros((Pb, Hb), jnp.float32)
    idx = jnp.clip(len_ref[...] - 1, 0, Lb - 1)
    for t in range(Lb):
        x = pe_ref[:, t, :]
        z = jnp.dot(jnp.concatenate([x, h], axis=1), wcat_ref[...],
                    preferred_element_type=jnp.float32) + bias_ref[...]
        gi = jax.nn.sigmoid(z[:, :Hb])
        gf = jax.nn.sigmoid(z[:, Hb:2 * Hb])
        gg = jnp.tanh(z[:, 2 * Hb:3 * Hb])
        go = jax.nn.sigmoid(z[:, 3 * Hb:])
        c = gf * c + gi * gg
        h = go * jnp.tanh(c)
        hsel = jnp.where(idx == t, h, hsel)
    pw_ref[...] = jax.nn.sigmoid(
        jnp.dot(hsel, wout_ref[...], preferred_element_type=jnp.float32)
        + bout_ref[...])


def _softmax_kernel(a_ref, gnn_ref, padj_ref, pemd_ref):
    i = pl.program_id(0)
    Bb, Nb = a_ref.shape
    a = a_ref[...]
    rows = jax.lax.broadcasted_iota(jnp.int32, (Bb, Nb), 0) + i * Bb
    cols = jax.lax.broadcasted_iota(jnp.int32, (Bb, Nb), 1)
    a = a + jnp.where(rows == cols, 1.0, 0.0)
    b = jnp.where(a > 0, a, jnp.float32(-9e15))
    m = jnp.max(b, axis=1, keepdims=True)
    e = jnp.exp(b - m)
    s = jnp.sum(e, axis=1, keepdims=True)
    p = e / s
    padj_ref[...] = p
    pemd_ref[...] = jnp.dot(p, gnn_ref[...], preferred_element_type=jnp.float32)


def _mlp_kernel(gnn_ref, pemd_ref, w1a_ref, w1b_ref, b1_ref, w2_ref, b2_ref,
                out_ref):
    h = jax.nn.relu(
        jnp.dot(gnn_ref[...], w1a_ref[...], preferred_element_type=jnp.float32)
        + jnp.dot(pemd_ref[...], w1b_ref[...],
                  preferred_element_type=jnp.float32)
        + b1_ref[...])
    logits = jnp.dot(h, w2_ref[...], preferred_element_type=jnp.float32) \
        + b2_ref[...]
    m = jnp.max(logits, axis=1, keepdims=True)
    lse = jnp.log(jnp.sum(jnp.exp(logits - m), axis=1, keepdims=True)) + m
    out_ref[...] = logits - lse


def kernel(features, adj, pairs, sub_paths, sub_path_length, W_pw, W_ih, W_hh,
           b_ih, b_hh, w_out, b_out, W1, b1, W2, b2):
    N_, F_ = features.shape
    D_ = W_pw.shape[1]
    P_, L_ = sub_paths.shape
    H_ = W_hh.shape[1]

    BRF = _pick_block(N_, 2048)
    emb0 = pl.pallas_call(
        _emb0_kernel,
        grid=(N_ // BRF,),
        in_specs=[pl.BlockSpec((BRF, F_), lambda i: (i, 0)),
                  pl.BlockSpec((F_, D_), lambda i: (0, 0))],
        out_specs=pl.BlockSpec((BRF, D_), lambda i: (i, 0)),
        out_shape=jax.ShapeDtypeStruct((N_, D_), jnp.float32),
    )(features, W_pw)

    BA = _pick_block(N_, 256)
    x1 = pl.pallas_call(
        _spmv1_kernel,
        grid=(N_ // BA,),
        in_specs=[pl.BlockSpec((BA, N_), lambda i: (i, 0)),
                  pl.BlockSpec((N_, D_), lambda i: (0, 0))],
        out_specs=pl.BlockSpec((BA, D_), lambda i: (i, 0)),
        out_shape=jax.ShapeDtypeStruct((N_, D_), jnp.float32),
    )(adj, emb0)
    gnn = pl.pallas_call(
        _spmv2_kernel,
        grid=(N_ // BA,),
        in_specs=[pl.BlockSpec((BA, N_), lambda i: (i, 0)),
                  pl.BlockSpec((N_, D_), lambda i: (0, 0)),
                  pl.BlockSpec((BA, D_), lambda i: (i, 0)),
                  pl.BlockSpec((BA, D_), lambda i: (i, 0))],
        out_specs=pl.BlockSpec((BA, D_), lambda i: (i, 0)),
        out_shape=jax.ShapeDtypeStruct((N_, D_), jnp.float32),
    )(adj, x1, emb0, x1)

    path_emb = gnn[sub_paths]
    wcat = jnp.concatenate([W_ih, W_hh], axis=1).T
    bias = (b_ih + b_hh)[None, :]
    wout = w_out[:, None]
    bout = b_out.reshape(1, 1)
    lens = sub_path_length.reshape(P_, 1).astype(jnp.int32)
    PC = _pick_block(P_, 2000)
    pw2 = pl.pallas_call(
        _lstm_kernel,
        grid=(P_ // PC,),
        in_specs=[pl.BlockSpec((PC, L_, D_), lambda i: (i, 0, 0)),
                  pl.BlockSpec((PC, 1), lambda i: (i, 0)),
                  pl.BlockSpec((D_ + H_, 4 * H_), lambda i: (0, 0)),
                  pl.BlockSpec((1, 4 * H_), lambda i: (0, 0)),
                  pl.BlockSpec((H_, 1), lambda i: (0, 0)),
                  pl.BlockSpec((1, 1), lambda i: (0, 0))],
        out_specs=pl.BlockSpec((PC, 1), lambda i: (i, 0)),
        out_shape=jax.ShapeDtypeStruct((P_, 1), jnp.float32),
    )(path_emb, lens, wcat, bias, wout, bout)
    pw = pw2[:, 0]

    A = jnp.zeros((N_, N_), jnp.float32).at[pairs[:, 0], pairs[:, 1]].add(pw)

    BS = _pick_block(N_, 200)
    padj, pemd = pl.pallas_call(
        _softmax_kernel,
        grid=(N_ // BS,),
        in_specs=[pl.BlockSpec((BS, N_), lambda i: (i, 0)),
                  pl.BlockSpec((N_, D_), lambda i: (0, 0))],
        out_specs=[pl.BlockSpec((BS, N_), lambda i: (i, 0)),
                   pl.BlockSpec((BS, D_), lambda i: (i, 0))],
        out_shape=[jax.ShapeDtypeStruct((N_, N_), jnp.float32),
                   jax.ShapeDtypeStruct((N_, D_), jnp.float32)],
    )(A, gnn)

    NH_ = W1.shape[1]
    NC_ = W2.shape[1]
    BM = _pick_block(N_, 2000)
    logits = pl.pallas_call(
        _mlp_kernel,
        grid=(N_ // BM,),
        in_specs=[pl.BlockSpec((BM, D_), lambda i: (i, 0)),
                  pl.BlockSpec((BM, D_), lambda i: (i, 0)),
                  pl.BlockSpec((D_, NH_), lambda i: (0, 0)),
                  pl.BlockSpec((D_, NH_), lambda i: (0, 0)),
                  pl.BlockSpec((1, NH_), lambda i: (0, 0)),
                  pl.BlockSpec((NH_, NC_), lambda i: (0, 0)),
                  pl.BlockSpec((1, NC_), lambda i: (0, 0))],
        out_specs=pl.BlockSpec((BM, NC_), lambda i: (i, 0)),
        out_shape=jax.ShapeDtypeStruct((N_, NC_), jnp.float32),
    )(gnn, pemd, W1[:D_], W1[D_:], b1[None, :], W2, b2[None, :])

    return (logits, padj)


# speedup vs baseline: 2.7813x; 1.7128x over previous
import functools

import jax
import jax.numpy as jnp
from jax import lax
from jax.experimental import pallas as pl
from jax.experimental.pallas import tpu as pltpu
from jax.experimental.pallas import tpu_sc as plsc


def _pick_block(n, target, mult=8):
    best = None
    for b in range(mult, min(n, target) + 1, mult):
        if n % b == 0:
            best = b
    return best if best is not None else n


def _emb0_kernel(f_ref, w_ref, o_ref):
    o_ref[...] = jax.nn.relu(
        jnp.dot(f_ref[...], w_ref[...], preferred_element_type=jnp.float32))


def _spmv1_kernel(adj_ref, e_ref, x1_ref):
    x1_ref[...] = jnp.dot(adj_ref[...], e_ref[...],
                          preferred_element_type=jnp.float32)


def _spmv2_kernel(adj_ref, x1f_ref, e_blk_ref, x1_blk_ref, gnn_ref):
    x2 = jnp.dot(adj_ref[...], x1f_ref[...], preferred_element_type=jnp.float32)
    gnn_ref[...] = (e_blk_ref[...] + x1_blk_ref[...] + x2) * (1.0 / 3.0)


def _lstm_kernel(pe_ref, len_ref, wcat_ref, bias_ref, wout_ref, bout_ref,
                 pw_ref):
    Pb, Lb, _ = pe_ref.shape
    Hb = wcat_ref.shape[1] // 4
    h = jnp.zeros((Pb, Hb), jnp.float32)
    c = jnp.zeros((Pb, Hb), jnp.float32)
    hsel = jnp.zeros((Pb, Hb), jnp.float32)
    idx = jnp.clip(len_ref[...] - 1, 0, Lb - 1)
    for t in range(Lb):
        x = pe_ref[:, t, :]
        z = jnp.dot(jnp.concatenate([x, h], axis=1), wcat_ref[...],
                    preferred_element_type=jnp.float32) + bias_ref[...]
        gi = jax.nn.sigmoid(z[:, :Hb])
        gf = jax.nn.sigmoid(z[:, Hb:2 * Hb])
        gg = jnp.tanh(z[:, 2 * Hb:3 * Hb])
        go = jax.nn.sigmoid(z[:, 3 * Hb:])
        c = gf * c + gi * gg
        h = go * jnp.tanh(c)
        hsel = jnp.where(idx == t, h, hsel)
    pw_ref[...] = jax.nn.sigmoid(
        jnp.dot(hsel, wout_ref[...], preferred_element_type=jnp.float32)
        + bout_ref[...])


def _softmax_kernel(a_ref, gnn_ref, padj_ref, pemd_ref):
    i = pl.program_id(0)
    Bb, Nb = a_ref.shape
    a = a_ref[...]
    rows = jax.lax.broadcasted_iota(jnp.int32, (Bb, Nb), 0) + i * Bb
    cols = jax.lax.broadcasted_iota(jnp.int32, (Bb, Nb), 1)
    a = a + jnp.where(rows == cols, 1.0, 0.0)
    b = jnp.where(a > 0, a, jnp.float32(-9e15))
    m = jnp.max(b, axis=1, keepdims=True)
    e = jnp.exp(b - m)
    s = jnp.sum(e, axis=1, keepdims=True)
    p = e / s
    padj_ref[...] = p
    pemd_ref[...] = jnp.dot(p, gnn_ref[...], preferred_element_type=jnp.float32)


def _mlp_kernel(gnn_ref, pemd_ref, w1a_ref, w1b_ref, b1_ref, w2_ref, b2_ref,
                out_ref):
    h = jax.nn.relu(
        jnp.dot(gnn_ref[...], w1a_ref[...], preferred_element_type=jnp.float32)
        + jnp.dot(pemd_ref[...], w1b_ref[...],
                  preferred_element_type=jnp.float32)
        + b1_ref[...])
    logits = jnp.dot(h, w2_ref[...], preferred_element_type=jnp.float32) \
        + b2_ref[...]
    m = jnp.max(logits, axis=1, keepdims=True)
    lse = jnp.log(jnp.sum(jnp.exp(logits - m), axis=1, keepdims=True)) + m
    out_ref[...] = logits - lse


def _sc_gather_rows(table, idx_flat):
    Nt, Dt = table.shape
    B = idx_flat.shape[0]
    NC, NS = 2, 16
    NW = NC * NS
    b_per_w = B // NW
    BQ = 600
    n_bat = b_per_w // BQ
    assert b_per_w * NW == B and n_bat * BQ == b_per_w and BQ % 8 == 0
    mesh = plsc.VectorSubcoreMesh(core_axis_name="c", subcore_axis_name="s")

    @functools.partial(
        pl.kernel, mesh=mesh,
        out_type=jax.ShapeDtypeStruct((B, Dt), jnp.float32),
        compiler_params=pltpu.CompilerParams(use_tc_tiling_on_sc=False),
        scratch_types=[
            pltpu.VMEM((BQ,), jnp.int32),
            pltpu.VMEM((BQ,), jnp.int32),
            pltpu.VMEM((BQ, Dt), jnp.float32),
            pltpu.VMEM((BQ, Dt), jnp.float32),
            pltpu.SemaphoreType.DMA,
            pltpu.SemaphoreType.DMA,
        ],
    )
    def k(table_hbm, idx_hbm, out_hbm, idx_a, idx_b, rows_a, rows_b,
          sem0, sem1):
        wid = lax.axis_index("s") * NC + lax.axis_index("c")
        base = wid * b_per_w

        def issue(g, idx_v, rows_v, sem):
            pltpu.sync_copy(idx_hbm.at[pl.ds(base + g * BQ, BQ)], idx_v)
            pltpu.async_copy(table_hbm.at[idx_v], rows_v, sem)

        def drain(g, idx_v, rows_v, sem):
            pltpu.make_async_copy(table_hbm.at[idx_v], rows_v, sem).wait()
            pltpu.sync_copy(rows_v, out_hbm.at[pl.ds(base + g * BQ, BQ)])

        issue(0, idx_a, rows_a, sem0)

        def body(i, carry):
            g = i * 2
            issue2 = g + 1 < n_bat

            @pl.when(issue2)
            def _():
                issue(g + 1, idx_b, rows_b, sem1)
            drain(g, idx_a, rows_a, sem0)

            @pl.when(g + 2 < n_bat)
            def _():
                issue(g + 2, idx_a, rows_a, sem0)

            @pl.when(issue2)
            def _():
                drain(g + 1, idx_b, rows_b, sem1)
            return carry

        lax.fori_loop(0, (n_bat + 1) // 2, body, 0)

    return k(table, idx_flat)


def kernel(features, adj, pairs, sub_paths, sub_path_length, W_pw, W_ih, W_hh,
           b_ih, b_hh, w_out, b_out, W1, b1, W2, b2):
    N_, F_ = features.shape
    D_ = W_pw.shape[1]
    P_, L_ = sub_paths.shape
    H_ = W_hh.shape[1]

    BRF = _pick_block(N_, 2048)
    emb0 = pl.pallas_call(
        _emb0_kernel,
        grid=(N_ // BRF,),
        in_specs=[pl.BlockSpec((BRF, F_), lambda i: (i, 0)),
                  pl.BlockSpec((F_, D_), lambda i: (0, 0))],
        out_specs=pl.BlockSpec((BRF, D_), lambda i: (i, 0)),
        out_shape=jax.ShapeDtypeStruct((N_, D_), jnp.float32),
    )(features, W_pw)

    BA = _pick_block(N_, 256)
    x1 = pl.pallas_call(
        _spmv1_kernel,
        grid=(N_ // BA,),
        in_specs=[pl.BlockSpec((BA, N_), lambda i: (i, 0)),
                  pl.BlockSpec((N_, D_), lambda i: (0, 0))],
        out_specs=pl.BlockSpec((BA, D_), lambda i: (i, 0)),
        out_shape=jax.ShapeDtypeStruct((N_, D_), jnp.float32),
    )(adj, emb0)
    gnn = pl.pallas_call(
        _spmv2_kernel,
        grid=(N_ // BA,),
        in_specs=[pl.BlockSpec((BA, N_), lambda i: (i, 0)),
                  pl.BlockSpec((N_, D_), lambda i: (0, 0)),
                  pl.BlockSpec((BA, D_), lambda i: (i, 0)),
                  pl.BlockSpec((BA, D_), lambda i: (i, 0))],
        out_specs=pl.BlockSpec((BA, D_), lambda i: (i, 0)),
        out_shape=jax.ShapeDtypeStruct((N_, D_), jnp.float32),
    )(adj, x1, emb0, x1)

    path_emb = _sc_gather_rows(gnn, sub_paths.reshape(-1)).reshape(P_, L_, D_)
    wcat = jnp.concatenate([W_ih, W_hh], axis=1).T
    bias = (b_ih + b_hh)[None, :]
    wout = w_out[:, None]
    bout = b_out.reshape(1, 1)
    lens = sub_path_length.reshape(P_, 1).astype(jnp.int32)
    PC = _pick_block(P_, 2000)
    pw2 = pl.pallas_call(
        _lstm_kernel,
        grid=(P_ // PC,),
        in_specs=[pl.BlockSpec((PC, L_, D_), lambda i: (i, 0, 0)),
                  pl.BlockSpec((PC, 1), lambda i: (i, 0)),
                  pl.BlockSpec((D_ + H_, 4 * H_), lambda i: (0, 0)),
                  pl.BlockSpec((1, 4 * H_), lambda i: (0, 0)),
                  pl.BlockSpec((H_, 1), lambda i: (0, 0)),
                  pl.BlockSpec((1, 1), lambda i: (0, 0))],
        out_specs=pl.BlockSpec((PC, 1), lambda i: (i, 0)),
        out_shape=jax.ShapeDtypeStruct((P_, 1), jnp.float32),
    )(path_emb, lens, wcat, bias, wout, bout)
    pw = pw2[:, 0]

    A = jnp.zeros((N_, N_), jnp.float32).at[pairs[:, 0], pairs[:, 1]].add(pw)

    BS = _pick_block(N_, 200)
    padj, pemd = pl.pallas_call(
        _softmax_kernel,
        grid=(N_ // BS,),
        in_specs=[pl.BlockSpec((BS, N_), lambda i: (i, 0)),
                  pl.BlockSpec((N_, D_), lambda i: (0, 0))],
        out_specs=[pl.BlockSpec((BS, N_), lambda i: (i, 0)),
                   pl.BlockSpec((BS, D_), lambda i: (i, 0))],
        out_shape=[jax.ShapeDtypeStruct((N_, N_), jnp.float32),
                   jax.ShapeDtypeStruct((N_, D_), jnp.float32)],
    )(A, gnn)

    NH_ = W1.shape[1]
    NC_ = W2.shape[1]
    BM = _pick_block(N_, 2000)
    logits = pl.pallas_call(
        _mlp_kernel,
        grid=(N_ // BM,),
        in_specs=[pl.BlockSpec((BM, D_), lambda i: (i, 0)),
                  pl.BlockSpec((BM, D_), lambda i: (i, 0)),
                  pl.BlockSpec((D_, NH_), lambda i: (0, 0)),
                  pl.BlockSpec((D_, NH_), lambda i: (0, 0)),
                  pl.BlockSpec((1, NH_), lambda i: (0, 0)),
                  pl.BlockSpec((NH_, NC_), lambda i: (0, 0)),
                  pl.BlockSpec((1, NC_), lambda i: (0, 0))],
        out_specs=pl.BlockSpec((BM, NC_), lambda i: (i, 0)),
        out_shape=jax.ShapeDtypeStruct((N_, NC_), jnp.float32),
    )(gnn, pemd, W1[:D_], W1[D_:], b1[None, :], W2, b2[None, :])

    return (logits, padj)
